# Initial kernel scaffold; baseline (speedup 1.0000x reference)
import functools

import jax
import jax.numpy as jnp
from jax import lax
from jax.experimental import pallas as pl
from jax.experimental.pallas import tpu as pltpu
from jax.experimental.pallas import tpu_sc as plsc

NC = 2
NS = 16
NW = NC * NS
CHUNK = 128
LANES = 16
D = 128


def _sc_mesh():
    return plsc.VectorSubcoreMesh(
        core_axis_name="c", subcore_axis_name="s", num_cores=NC, num_subcores=NS
    )


def _make_degree_kernel(n_pad, G):
    zr = n_pad // NS

    @functools.partial(
        pl.kernel,
        out_type=jax.ShapeDtypeStruct((NC, n_pad, LANES), jnp.float32),
        mesh=_sc_mesh(),
        scratch_types=[
            pltpu.VMEM((G, CHUNK), jnp.int32),
            pltpu.VMEM((CHUNK, LANES), jnp.float32),
            pltpu.VMEM_SHARED((n_pad, LANES), jnp.float32),
            pltpu.SemaphoreType.DMA,
        ],
    )
    def deg_kernel(dst3, ones_hbm, zeros_hbm, hist_out, idx_v, ones_v, hist_sh, sem):
        c = lax.axis_index("c")
        s = lax.axis_index("s")
        w = c * NS + s
        pltpu.async_copy(dst3.at[w], idx_v, sem).wait()
        pltpu.async_copy(ones_hbm, ones_v, sem).wait()
        pltpu.sync_copy(zeros_hbm, hist_sh.at[pl.ds(s * zr, zr)])
        plsc.subcore_barrier()

        def body(g, carry):
            pltpu.sync_copy(ones_v, hist_sh.at[idx_v.at[g]], add=True)
            return carry

        lax.fori_loop(0, G, body, 0)
        plsc.subcore_barrier()
        pltpu.sync_copy(hist_sh.at[pl.ds(s * zr, zr)],
                        hist_out.at[c, pl.ds(s * zr, zr)])

    return deg_kernel


def _make_agg_kernel(n_pad, G):
    zr = n_pad // NS

    @functools.partial(
        pl.kernel,
        out_type=jax.ShapeDtypeStruct((NC, n_pad, D), jnp.float32),
        mesh=_sc_mesh(),
        scratch_types=[
            pltpu.VMEM((G, CHUNK), jnp.int32),
            pltpu.VMEM((G, CHUNK), jnp.int32),
            pltpu.VMEM((CHUNK, D), jnp.float32),
            pltpu.VMEM_SHARED((n_pad, D), jnp.float32),
            pltpu.SemaphoreType.DMA,
        ],
    )
    def agg_kernel(y_hbm, src3, dst3, out_hbm, sidx_v, didx_v, rows_v, acc_sh, sem):
        c = lax.axis_index("c")
        s = lax.axis_index("s")
        w = c * NS + s
        pltpu.async_copy(src3.at[w], sidx_v, sem).wait()
        pltpu.async_copy(dst3.at[w], didx_v, sem).wait()

        zero16 = jnp.zeros((LANES,), jnp.float32)

        def zbody(i, carry):
            rows_v[i // 8, pl.ds((i % 8) * LANES, LANES)] = zero16
            return carry

        lax.fori_loop(0, CHUNK * (D // LANES), zbody, 0)

        def zcp(k, carry):
            pltpu.sync_copy(rows_v, acc_sh.at[pl.ds(s * zr + k * CHUNK, CHUNK)])
            return carry

        lax.fori_loop(0, zr // CHUNK, zcp, 0)
        plsc.subcore_barrier()

        def body(g, carry):
            pltpu.async_copy(y_hbm.at[sidx_v.at[g]], rows_v, sem).wait()
            pltpu.sync_copy(rows_v, acc_sh.at[didx_v.at[g]], add=True)
            return carry

        lax.fori_loop(0, G, body, 0)
        plsc.subcore_barrier()
        pltpu.sync_copy(acc_sh.at[pl.ds(s * zr, zr)],
                        out_hbm.at[c, pl.ds(s * zr, zr)])

    return agg_kernel


def _make_scale_kernel(n_pad):
    R = 1024

    def body(h0_ref, h1_ref, x_ref, y_ref, dinv_ref):
        deg = h0_ref[:, :1] + h1_ref[:, :1] + 1.0
        dinv = lax.rsqrt(deg)
        y_ref[...] = x_ref[...] * dinv
        dinv_ref[...] = dinv

    return pl.pallas_call(
        body,
        grid=(n_pad // R,),
        in_specs=[
            pl.BlockSpec((R, LANES), lambda i: (i, 0)),
            pl.BlockSpec((R, LANES), lambda i: (i, 0)),
            pl.BlockSpec((R, D), lambda i: (i, 0)),
        ],
        out_specs=[
            pl.BlockSpec((R, D), lambda i: (i, 0)),
            pl.BlockSpec((R, 1), lambda i: (i, 0)),
        ],
        out_shape=[
            jax.ShapeDtypeStruct((n_pad, D), jnp.float32),
            jax.ShapeDtypeStruct((n_pad, 1), jnp.float32),
        ],
    )


def _make_dense_kernel(n_pad):
    R = 1024

    def body(acc0_ref, acc1_ref, y_ref, dinv_ref, alpha_ref,
             wcat_ref, bcat_ref, wf_ref, bf_ref, wc_ref, bc_ref, out_ref):
        agg = dinv_ref[...] * (acc0_ref[...] + acc1_ref[...] + y_ref[...])
        hcat = jnp.dot(agg, wcat_ref[...], preferred_element_type=jnp.float32)
        hcat = hcat + bcat_ref[...]
        a = alpha_ref[...]
        h = a * hcat[:, :D] + (1.0 - a) * hcat[:, D:]
        h = jnp.maximum(h, 0.0)
        h2 = jnp.dot(h, wf_ref[...], preferred_element_type=jnp.float32)
        h2 = jnp.maximum(h2 + bf_ref[...], 0.0)
        out_ref[...] = (
            jnp.dot(h2, wc_ref[...], preferred_element_type=jnp.float32)
            + bc_ref[...]
        )

    full = lambda i: (0, 0)
    return pl.pallas_call(
        body,
        grid=(n_pad // R,),
        in_specs=[
            pl.BlockSpec((R, D), lambda i: (i, 0)),
            pl.BlockSpec((R, D), lambda i: (i, 0)),
            pl.BlockSpec((R, D), lambda i: (i, 0)),
            pl.BlockSpec((R, 1), lambda i: (i, 0)),
            pl.BlockSpec((R, 1), lambda i: (i, 0)),
            pl.BlockSpec((D, 2 * D), full),
            pl.BlockSpec((1, 2 * D), full),
            pl.BlockSpec((D, 64), full),
            pl.BlockSpec((1, 64), full),
            pl.BlockSpec((64, D), full),
            pl.BlockSpec((1, D), full),
        ],
        out_specs=pl.BlockSpec((R, D), lambda i: (i, 0)),
        out_shape=jax.ShapeDtypeStruct((n_pad, D), jnp.float32),
    )


def kernel(x, edge_index, h_node, W_homo, b_homo, W_het, b_het, Wf, bf, Wc, bc):
    N = x.shape[0]
    E = edge_index.shape[1]
    n_pad = ((N + 1) + NS * CHUNK - 1) // (NS * CHUNK) * (NS * CHUNK)
    G = (E + NW * CHUNK - 1) // (NW * CHUNK)
    e_pad = G * NW * CHUNK

    src = edge_index[0].astype(jnp.int32)
    dst = edge_index[1].astype(jnp.int32)
    pad_idx = jnp.full((e_pad - E,), N, dtype=jnp.int32)
    src3 = jnp.concatenate([src, pad_idx]).reshape(NW, G, CHUNK)
    dst3 = jnp.concatenate([dst, pad_idx]).reshape(NW, G, CHUNK)

    ones_h = jnp.ones((CHUNK, LANES), jnp.float32)
    zeros_h = jnp.zeros((n_pad // NS, LANES), jnp.float32)
    hist = _make_degree_kernel(n_pad, G)(dst3, ones_h, zeros_h)

    x_pad = jnp.zeros((n_pad, D), jnp.float32).at[:N].set(x)
    y_pad, dinv = _make_scale_kernel(n_pad)(hist[0], hist[1], x_pad)

    acc = _make_agg_kernel(n_pad, G)(y_pad, src3, dst3)

    alpha_pad = jnp.zeros((n_pad, 1), jnp.float32).at[:N, 0].set(1.0 - h_node)
    wcat = jnp.concatenate([W_homo, W_het], axis=1)
    bcat = jnp.concatenate([b_homo, b_het])[None, :]
    wc_pad = jnp.zeros((64, D), jnp.float32).at[:, : Wc.shape[1]].set(Wc)
    bc_pad = jnp.zeros((1, D), jnp.float32).at[0, : Wc.shape[1]].set(bc)

    out = _make_dense_kernel(n_pad)(
        acc[0], acc[1], y_pad, dinv, alpha_pad,
        wcat, bcat, Wf, bf[None, :], wc_pad, bc_pad,
    )
    return out[:N, : Wc.shape[1]]


# speedup vs baseline: 17.7100x; 17.7100x over previous
import functools

import jax
import jax.numpy as jnp
from jax import lax
from jax.experimental import pallas as pl
from jax.experimental.pallas import tpu as pltpu
from jax.experimental.pallas import tpu_sc as plsc

NC = 2
NS = 16
NW = NC * NS
CHUNK = 128
B_CH = 16
LANES = 16
D = 128


def _sc_mesh():
    return plsc.VectorSubcoreMesh(
        core_axis_name="c", subcore_axis_name="s", num_cores=NC, num_subcores=NS
    )


def _fill_rows(ref, value):
    vec = jnp.full((LANES,), value, jnp.float32)

    def body(i, carry):
        ref[i // (D // LANES), pl.ds((i % (D // LANES)) * LANES, LANES)] = vec
        return carry

    lax.fori_loop(0, CHUNK * (D // LANES), body, 0)


def _make_degree_kernel(n_pad, G):
    zr = n_pad // NS
    WINDOW = 4

    @functools.partial(
        pl.kernel,
        out_type=jax.ShapeDtypeStruct((NC, n_pad, D), jnp.float32),
        mesh=_sc_mesh(),
        scratch_types=[
            pltpu.VMEM((G, CHUNK), jnp.int32),
            pltpu.VMEM((CHUNK, D), jnp.float32),
            pltpu.VMEM_SHARED((n_pad, D), jnp.float32),
            pltpu.SemaphoreType.DMA,
            pltpu.SemaphoreType.DMA,
        ],
    )
    def deg_kernel(dst3, hist_out, idx_v, ones_v, hist_sh, semi, sems):
        c = lax.axis_index("c")
        s = lax.axis_index("s")
        w = c * NS + s
        pltpu.async_copy(dst3.at[w], idx_v, semi)
        _fill_rows(ones_v, 0.0)

        def zcp(k, carry):
            pltpu.sync_copy(ones_v, hist_sh.at[pl.ds(s * zr + k * CHUNK, CHUNK)])
            return carry

        lax.fori_loop(0, zr // CHUNK, zcp, 0)
        _fill_rows(ones_v, 1.0)
        pltpu.make_async_copy(dst3.at[w], idx_v, semi).wait()
        plsc.subcore_barrier()

        def body(g, carry):
            pltpu.async_copy(ones_v, hist_sh.at[idx_v.at[g]], sems, add=True)

            @pl.when(g >= WINDOW)
            def _():
                pltpu.make_async_copy(ones_v, hist_sh.at[idx_v.at[g]], sems).wait()

            return carry

        lax.fori_loop(0, G, body, 0)

        def drain(k, carry):
            pltpu.make_async_copy(ones_v, hist_sh.at[idx_v.at[0]], sems).wait()
            return carry

        lax.fori_loop(0, WINDOW, drain, 0)
        plsc.subcore_barrier()
        pltpu.sync_copy(hist_sh.at[pl.ds(s * zr, zr)],
                        hist_out.at[c, pl.ds(s * zr, zr)])

    return deg_kernel


def _make_agg_kernel(n_pad, G):
    zr = n_pad // NS
    NB = G // B_CH
    assert G % B_CH == 0

    @functools.partial(
        pl.kernel,
        out_type=jax.ShapeDtypeStruct((NC, n_pad, D), jnp.float32),
        mesh=_sc_mesh(),
        scratch_types=[
            pltpu.VMEM((B_CH, CHUNK), jnp.int32),
            pltpu.VMEM((B_CH, CHUNK), jnp.int32),
            pltpu.VMEM((CHUNK, D), jnp.float32),
            pltpu.VMEM((CHUNK, D), jnp.float32),
            pltpu.VMEM_SHARED((n_pad, D), jnp.float32),
            pltpu.SemaphoreType.DMA,
            pltpu.SemaphoreType.DMA,
            pltpu.SemaphoreType.DMA,
        ],
    )
    def agg_kernel(y_hbm, src3, dst3, out_hbm,
                   sidx_v, didx_v, rows0, rows1, acc_sh, semi, sem0, sem1):
        c = lax.axis_index("c")
        s = lax.axis_index("s")
        w = c * NS + s

        _fill_rows(rows0, 0.0)

        def zcp(k, carry):
            pltpu.sync_copy(rows0, acc_sh.at[pl.ds(s * zr + k * CHUNK, CHUNK)])
            return carry

        lax.fori_loop(0, zr // CHUNK, zcp, 0)
        plsc.subcore_barrier()

        def block(b, carry):
            base = b * B_CH
            pltpu.async_copy(src3.at[w, pl.ds(base, B_CH)], sidx_v, semi)
            pltpu.async_copy(dst3.at[w, pl.ds(base, B_CH)], didx_v, sem0)
            pltpu.make_async_copy(src3.at[w, pl.ds(base, B_CH)], sidx_v, semi).wait()
            pltpu.make_async_copy(dst3.at[w, pl.ds(base, B_CH)], didx_v, sem0).wait()
            pltpu.async_copy(y_hbm.at[sidx_v.at[0]], rows0, sem0)

            def pair(h, carry):
                j0 = h * 2
                pltpu.async_copy(y_hbm.at[sidx_v.at[j0 + 1]], rows1, sem1)
                pltpu.make_async_copy(y_hbm.at[sidx_v.at[j0]], rows0, sem0).wait()
                pltpu.sync_copy(rows0, acc_sh.at[didx_v.at[j0]], add=True)

                @pl.when(j0 + 2 < B_CH)
                def _():
                    pltpu.async_copy(y_hbm.at[sidx_v.at[j0 + 2]], rows0, sem0)

                pltpu.make_async_copy(y_hbm.at[sidx_v.at[j0 + 1]], rows1, sem1).wait()
                pltpu.sync_copy(rows1, acc_sh.at[didx_v.at[j0 + 1]], add=True)
                return carry

            lax.fori_loop(0, B_CH // 2, pair, 0)
            return carry

        lax.fori_loop(0, NB, block, 0)
        plsc.subcore_barrier()
        pltpu.sync_copy(acc_sh.at[pl.ds(s * zr, zr)],
                        out_hbm.at[c, pl.ds(s * zr, zr)])

    return agg_kernel


def _make_scale_kernel(n_pad):
    R = 1024

    def body(h0_ref, h1_ref, x_ref, y_ref, dinv_ref):
        deg = h0_ref[:, :1] + h1_ref[:, :1] + 1.0
        dinv = lax.rsqrt(deg)
        y_ref[...] = x_ref[...] * dinv
        dinv_ref[...] = dinv

    return pl.pallas_call(
        body,
        grid=(n_pad // R,),
        in_specs=[
            pl.BlockSpec((R, D), lambda i: (i, 0)),
            pl.BlockSpec((R, D), lambda i: (i, 0)),
            pl.BlockSpec((R, D), lambda i: (i, 0)),
        ],
        out_specs=[
            pl.BlockSpec((R, D), lambda i: (i, 0)),
            pl.BlockSpec((R, 1), lambda i: (i, 0)),
        ],
        out_shape=[
            jax.ShapeDtypeStruct((n_pad, D), jnp.float32),
            jax.ShapeDtypeStruct((n_pad, 1), jnp.float32),
        ],
    )


def _make_dense_kernel(n_pad):
    R = 1024

    def body(acc0_ref, acc1_ref, y_ref, dinv_ref, alpha_ref,
             wcat_ref, bcat_ref, wf_ref, bf_ref, wc_ref, bc_ref, out_ref):
        agg = dinv_ref[...] * (acc0_ref[...] + acc1_ref[...] + y_ref[...])
        hcat = jnp.dot(agg, wcat_ref[...], preferred_element_type=jnp.float32)
        hcat = hcat + bcat_ref[...]
        a = alpha_ref[...]
        h = a * hcat[:, :D] + (1.0 - a) * hcat[:, D:]
        h = jnp.maximum(h, 0.0)
        h2 = jnp.dot(h, wf_ref[...], preferred_element_type=jnp.float32)
        h2 = jnp.maximum(h2 + bf_ref[...], 0.0)
        out_ref[...] = (
            jnp.dot(h2, wc_ref[...], preferred_element_type=jnp.float32)
            + bc_ref[...]
        )

    full = lambda i: (0, 0)
    return pl.pallas_call(
        body,
        grid=(n_pad // R,),
        in_specs=[
            pl.BlockSpec((R, D), lambda i: (i, 0)),
            pl.BlockSpec((R, D), lambda i: (i, 0)),
            pl.BlockSpec((R, D), lambda i: (i, 0)),
            pl.BlockSpec((R, 1), lambda i: (i, 0)),
            pl.BlockSpec((R, 1), lambda i: (i, 0)),
            pl.BlockSpec((D, 2 * D), full),
            pl.BlockSpec((1, 2 * D), full),
            pl.BlockSpec((D, 64), full),
            pl.BlockSpec((1, 64), full),
            pl.BlockSpec((64, D), full),
            pl.BlockSpec((1, D), full),
        ],
        out_specs=pl.BlockSpec((R, D), lambda i: (i, 0)),
        out_shape=jax.ShapeDtypeStruct((n_pad, D), jnp.float32),
    )


def kernel(x, edge_index, h_node, W_homo, b_homo, W_het, b_het, Wf, bf, Wc, bc):
    N = x.shape[0]
    E = edge_index.shape[1]
    n_pad = ((N + 1) + NS * CHUNK - 1) // (NS * CHUNK) * (NS * CHUNK)
    G = (E + NW * CHUNK * B_CH - 1) // (NW * CHUNK * B_CH) * B_CH
    e_pad = G * NW * CHUNK

    src = edge_index[0].astype(jnp.int32)
    dst = edge_index[1].astype(jnp.int32)
    pad_idx = jnp.full((e_pad - E,), N, dtype=jnp.int32)
    src3 = jnp.concatenate([src, pad_idx]).reshape(NW, G, CHUNK)
    dst3 = jnp.concatenate([dst, pad_idx]).reshape(NW, G, CHUNK)

    hist = _make_degree_kernel(n_pad, G)(dst3)

    x_pad = jnp.zeros((n_pad, D), jnp.float32).at[:N].set(x)
    y_pad, dinv = _make_scale_kernel(n_pad)(hist[0], hist[1], x_pad)

    acc = _make_agg_kernel(n_pad, G)(y_pad, src3, dst3)

    alpha_pad = jnp.zeros((n_pad, 1), jnp.float32).at[:N, 0].set(1.0 - h_node)
    wcat = jnp.concatenate([W_homo, W_het], axis=1)
    bcat = jnp.concatenate([b_homo, b_het])[None, :]
    wc_pad = jnp.zeros((64, D), jnp.float32).at[:, : Wc.shape[1]].set(Wc)
    bc_pad = jnp.zeros((1, D), jnp.float32).at[0, : Wc.shape[1]].set(bc)

    out = _make_dense_kernel(n_pad)(
        acc[0], acc[1], y_pad, dinv, alpha_pad,
        wcat, bcat, Wf, bf[None, :], wc_pad, bc_pad,
    )
    return out[:N, : Wc.shape[1]]
